# Initial kernel scaffold; baseline (speedup 1.0000x reference)
import jax
import jax.numpy as jnp
from jax.experimental import pallas as pl


def kernel(x, edge_index, W1, b1, W2, b2, W3, b3):
    raise NotImplementedError("write your pallas kernel here")


# speedup vs baseline: 10.0046x; 10.0046x over previous
import functools

import jax
import jax.numpy as jnp
from jax import lax
from jax.experimental import pallas as pl
from jax.experimental.pallas import tpu as pltpu
from jax.experimental.pallas import tpu_sc as plsc

_NC = 2
_NS = 16
_NW = _NC * _NS
_B = 128
_RB = 1000


def _cdiv(a, b):
    return (a + b - 1) // b


def _make_deg_kernel(n_pad, nb):
    mesh = plsc.VectorSubcoreMesh(core_axis_name="c", subcore_axis_name="s")
    rpt = n_pad // _NS
    nz = rpt // _B

    @functools.partial(
        pl.kernel,
        out_type=jax.ShapeDtypeStruct((_NC, n_pad, 8), jnp.float32),
        mesh=mesh,
        compiler_params=pltpu.CompilerParams(use_tc_tiling_on_sc=False),
        scratch_types=[
            pltpu.VMEM((nb, _B), jnp.int32),
            pltpu.VMEM((_B, 8), jnp.float32),
            pltpu.VMEM_SHARED((n_pad, 8), jnp.float32),
        ],
    )
    def deg_kernel(dstp, ones_hbm, zeros_hbm, out, idxd, val_v, acc):
        c = lax.axis_index("c")
        s = lax.axis_index("s")
        t = c * _NS + s
        pltpu.sync_copy(dstp.at[t], idxd)
        pltpu.sync_copy(zeros_hbm, val_v)
        for k in range(nz):
            pltpu.sync_copy(val_v, acc.at[pl.ds(s * rpt + k * _B, _B)])
        pltpu.sync_copy(ones_hbm, val_v)
        plsc.subcore_barrier()

        @pl.loop(0, nb)
        def _(j):
            pltpu.sync_copy(val_v, acc.at[idxd.at[j]], add=True)

        plsc.subcore_barrier()
        for k in range(nz):
            off = s * rpt + k * _B
            pltpu.sync_copy(acc.at[pl.ds(off, _B)], out.at[c, pl.ds(off, _B)])

    return deg_kernel


def _make_scatter_kernel(n, n_pad, nb, d):
    mesh = plsc.VectorSubcoreMesh(core_axis_name="c", subcore_axis_name="s")
    rpt = n_pad // _NS
    nz = rpt // _B

    @functools.partial(
        pl.kernel,
        out_type=jax.ShapeDtypeStruct((_NC, n_pad, d), jnp.float32),
        mesh=mesh,
        compiler_params=pltpu.CompilerParams(use_tc_tiling_on_sc=False),
        scratch_types=[
            pltpu.VMEM((nb, _B), jnp.int32),
            pltpu.VMEM((nb, _B), jnp.int32),
            pltpu.VMEM((_B, d), jnp.float32),
            pltpu.VMEM((_B, d), jnp.float32),
            pltpu.VMEM_SHARED((n_pad, d), jnp.float32),
            pltpu.SemaphoreType.DMA,
            pltpu.SemaphoreType.DMA,
        ],
    )
    def scat_kernel(g_hbm, srcp, dstp, zeros_hbm, out,
                    idxs, idxd, buf0, buf1, acc, sem0, sem1):
        c = lax.axis_index("c")
        s = lax.axis_index("s")
        t = c * _NS + s
        pltpu.sync_copy(srcp.at[t], idxs)
        pltpu.sync_copy(dstp.at[t], idxd)
        pltpu.sync_copy(zeros_hbm, buf0)
        for k in range(nz):
            pltpu.sync_copy(buf0, acc.at[pl.ds(s * rpt + k * _B, _B)])
        pltpu.async_copy(g_hbm.at[idxs.at[0]], buf0, sem0)
        pltpu.async_copy(g_hbm.at[idxs.at[1]], buf1, sem1)
        plsc.subcore_barrier()

        @pl.loop(0, nb - 2, step=2)
        def _(j):
            pltpu.make_async_copy(g_hbm.at[idxs.at[j]], buf0, sem0).wait()
            pltpu.sync_copy(buf0, acc.at[idxd.at[j]], add=True)
            pltpu.async_copy(g_hbm.at[idxs.at[j + 2]], buf0, sem0)
            pltpu.make_async_copy(g_hbm.at[idxs.at[j + 1]], buf1, sem1).wait()
            pltpu.sync_copy(buf1, acc.at[idxd.at[j + 1]], add=True)
            pltpu.async_copy(g_hbm.at[idxs.at[j + 3]], buf1, sem1)

        pltpu.make_async_copy(g_hbm.at[idxs.at[nb - 2]], buf0, sem0).wait()
        pltpu.sync_copy(buf0, acc.at[idxd.at[nb - 2]], add=True)
        pltpu.make_async_copy(g_hbm.at[idxs.at[nb - 1]], buf1, sem1).wait()
        pltpu.sync_copy(buf1, acc.at[idxd.at[nb - 1]], add=True)

        plsc.subcore_barrier()
        for k in range(nz):
            off = s * rpt + k * _B
            pltpu.sync_copy(acc.at[pl.ds(off, _B)], out.at[c, pl.ds(off, _B)])

    return scat_kernel


def _dinv_block(dp_ref):
    deg = dp_ref[0, :, 0:1] + dp_ref[1, :, 0:1] + 1.0
    return lax.rsqrt(deg)


def _in_body(dp_ref, x_ref, w_ref, o_ref):
    dinv = _dinv_block(dp_ref)
    o_ref[...] = dinv * jnp.dot(x_ref[...], w_ref[...],
                                preferred_element_type=jnp.float32)


def _mid_body(dp_ref, sp_ref, g_ref, b_ref, w_ref, o_ref):
    dinv = _dinv_block(dp_ref)
    sfull = sp_ref[0] + sp_ref[1] + g_ref[...]
    h = jnp.maximum(dinv * sfull + b_ref[...], 0.0)
    o_ref[...] = dinv * jnp.dot(h, w_ref[...],
                                preferred_element_type=jnp.float32)


def _mid2_body(dp_ref, spa_ref, spb_ref, g_ref, b_ref, w_ref, o_ref):
    dinv = _dinv_block(dp_ref)
    sfull = jnp.concatenate(
        [spa_ref[0] + spa_ref[1], spb_ref[0] + spb_ref[1]], axis=1)
    sfull = sfull + g_ref[...]
    h = jnp.maximum(dinv * sfull + b_ref[...], 0.0)
    o_ref[...] = dinv * jnp.dot(h, w_ref[...],
                                preferred_element_type=jnp.float32)


def _out_body(dp_ref, sp_ref, g_ref, b_ref, o_ref):
    dinv = _dinv_block(dp_ref)
    sfull = sp_ref[0] + sp_ref[1] + g_ref[...]
    o_ref[...] = dinv * sfull + b_ref[...]


def _dp_spec():
    return pl.BlockSpec((2, _RB, 8), lambda i: (0, i, 0))


def _rows(d):
    return pl.BlockSpec((_RB, d), lambda i: (i, 0))


def _full(shape):
    nd = len(shape)
    return pl.BlockSpec(shape, lambda i: (0,) * nd)


def _tc_in(n, deg_parts, x, w):
    d_in, d_out = w.shape
    return pl.pallas_call(
        _in_body,
        grid=(n // _RB,),
        in_specs=[_dp_spec(), _rows(d_in), _full((d_in, d_out))],
        out_specs=_rows(d_out),
        out_shape=jax.ShapeDtypeStruct((n, d_out), jnp.float32),
    )(deg_parts, x, w)


def _tc_mid(n, deg_parts, s_parts, g, b, w):
    d_in, d_out = w.shape
    return pl.pallas_call(
        _mid_body,
        grid=(n // _RB,),
        in_specs=[_dp_spec(),
                  pl.BlockSpec((2, _RB, d_in), lambda i: (0, i, 0)),
                  _rows(d_in), _full((1, d_in)), _full((d_in, d_out))],
        out_specs=_rows(d_out),
        out_shape=jax.ShapeDtypeStruct((n, d_out), jnp.float32),
    )(deg_parts, s_parts, g, b.reshape(1, d_in), w)


def _tc_mid2(n, deg_parts, s_a, s_b, g, b, w):
    d_in, d_out = w.shape
    dh = d_in // 2
    half = pl.BlockSpec((2, _RB, dh), lambda i: (0, i, 0))
    return pl.pallas_call(
        _mid2_body,
        grid=(n // _RB,),
        in_specs=[_dp_spec(), half, half,
                  _rows(d_in), _full((1, d_in)), _full((d_in, d_out))],
        out_specs=_rows(d_out),
        out_shape=jax.ShapeDtypeStruct((n, d_out), jnp.float32),
    )(deg_parts, s_a, s_b, g, b.reshape(1, d_in), w)


def _tc_out(n, deg_parts, s_parts, g, b):
    d = g.shape[1]
    return pl.pallas_call(
        _out_body,
        grid=(n // _RB,),
        in_specs=[_dp_spec(),
                  pl.BlockSpec((2, _RB, d), lambda i: (0, i, 0)),
                  _rows(d), _full((1, d))],
        out_specs=_rows(d),
        out_shape=jax.ShapeDtypeStruct((n, d), jnp.float32),
    )(deg_parts, s_parts, g, b.reshape(1, d))


def kernel(x, edge_index, W1, b1, W2, b2, W3, b3):
    n, d_in = x.shape
    e = edge_index.shape[1]

    n_pad = _cdiv(n, _NS * _B) * _NS * _B
    nb = _cdiv(e, _NW * _B)
    nb += nb % 2
    e_pad = _NW * nb * _B

    src = edge_index[0]
    dst = edge_index[1]
    pad = e_pad - e
    srcp = jnp.concatenate(
        [src, jnp.zeros((pad,), jnp.int32)]).reshape(_NW, nb, _B)
    dstp = jnp.concatenate(
        [dst, jnp.full((pad,), n, jnp.int32)]).reshape(_NW, nb, _B)

    ones8 = jnp.ones((_B, 8), jnp.float32)
    zeros8 = jnp.zeros((_B, 8), jnp.float32)
    dh = W1.shape[1] // 2
    z64 = jnp.zeros((_B, dh), jnp.float32)

    deg_parts = _make_deg_kernel(n_pad, nb)(dstp, ones8, zeros8)

    scat64 = _make_scatter_kernel(n, n_pad, nb, dh)
    g1 = _tc_in(n, deg_parts, x, W1)
    s1a = scat64(g1[:, :dh], srcp, dstp, z64)
    s1b = scat64(g1[:, dh:], srcp, dstp, z64)
    g2 = _tc_mid2(n, deg_parts, s1a, s1b, g1, b1, W2)
    s2 = scat64(g2, srcp, dstp, z64)
    g3 = _tc_mid(n, deg_parts, s2, g2, b2, W3)
    s3 = scat64(g3, srcp, dstp, z64)
    return _tc_out(n, deg_parts, s3, g3, b3)


# speedup vs baseline: 17.7411x; 1.7733x over previous
import functools

import jax
import jax.numpy as jnp
from jax import lax
from jax.experimental import pallas as pl
from jax.experimental.pallas import tpu as pltpu
from jax.experimental.pallas import tpu_sc as plsc

_NC = 2
_NS = 16
_NW = _NC * _NS
_B = 128
_RB = 1000


def _cdiv(a, b):
    return (a + b - 1) // b


def _make_deg_kernel(n_pad, tb):
    mesh = plsc.VectorSubcoreMesh(core_axis_name="c", subcore_axis_name="s")
    rpt = n_pad // _NS
    nz = rpt // _B
    nb = tb // _NW

    @functools.partial(
        pl.kernel,
        out_type=jax.ShapeDtypeStruct((_NC, n_pad, 8), jnp.float32),
        mesh=mesh,
        compiler_params=pltpu.CompilerParams(use_tc_tiling_on_sc=False),
        scratch_types=[
            pltpu.VMEM((nb, _B), jnp.int32),
            pltpu.VMEM((_B, 8), jnp.float32),
            pltpu.VMEM_SHARED((n_pad, 8), jnp.float32),
        ],
    )
    def deg_kernel(dstp, ones_hbm, zeros_hbm, out, idxd, val_v, acc):
        c = lax.axis_index("c")
        s = lax.axis_index("s")
        t = c * _NS + s
        pltpu.sync_copy(dstp.at[pl.ds(t * nb, nb)], idxd)
        pltpu.sync_copy(zeros_hbm, val_v)
        for k in range(nz):
            pltpu.sync_copy(val_v, acc.at[pl.ds(s * rpt + k * _B, _B)])
        pltpu.sync_copy(ones_hbm, val_v)
        plsc.subcore_barrier()

        @pl.loop(0, nb)
        def _(j):
            pltpu.sync_copy(val_v, acc.at[idxd.at[j]], add=True)

        plsc.subcore_barrier()
        for k in range(nz):
            off = s * rpt + k * _B
            pltpu.sync_copy(acc.at[pl.ds(off, _B)], out.at[c, pl.ds(off, _B)])

    return deg_kernel


def _make_scatter_kernel(n, n_pad, nb0, nb1, d):
    mesh = plsc.VectorSubcoreMesh(core_axis_name="c", subcore_axis_name="s")
    rpt = n_pad // _NS
    nz = rpt // _B

    @functools.partial(
        pl.kernel,
        out_type=jax.ShapeDtypeStruct((_NC, n_pad, d), jnp.float32),
        mesh=mesh,
        compiler_params=pltpu.CompilerParams(use_tc_tiling_on_sc=False),
        scratch_types=[
            pltpu.VMEM((max(nb0, nb1), _B), jnp.int32),
            pltpu.VMEM((max(nb0, nb1), _B), jnp.int32),
            pltpu.VMEM((_B, d), jnp.float32),
            pltpu.VMEM((_B, d), jnp.float32),
            pltpu.VMEM_SHARED((n_pad, d), jnp.float32),
            pltpu.SemaphoreType.DMA,
            pltpu.SemaphoreType.DMA,
        ],
    )
    def scat_kernel(g_hbm, srcp, dstp, zeros_hbm, out,
                    idxs, idxd, buf0, buf1, acc, sem0, sem1):
        c = lax.axis_index("c")
        s = lax.axis_index("s")
        pltpu.sync_copy(zeros_hbm, buf0)
        for k in range(nz):
            pltpu.sync_copy(buf0, acc.at[pl.ds(s * rpt + k * _B, _B)])

        def run(nb, base):
            pltpu.sync_copy(srcp.at[pl.ds(base, nb)], idxs.at[pl.ds(0, nb)])
            pltpu.sync_copy(dstp.at[pl.ds(base, nb)], idxd.at[pl.ds(0, nb)])
            pltpu.async_copy(g_hbm.at[idxs.at[0]], buf0, sem0)
            pltpu.async_copy(g_hbm.at[idxs.at[1]], buf1, sem1)
            plsc.subcore_barrier()

            @pl.loop(0, nb - 2, step=2)
            def _(j):
                pltpu.make_async_copy(g_hbm.at[idxs.at[j]], buf0, sem0).wait()
                pltpu.sync_copy(buf0, acc.at[idxd.at[j]], add=True)
                pltpu.async_copy(g_hbm.at[idxs.at[j + 2]], buf0, sem0)
                pltpu.make_async_copy(
                    g_hbm.at[idxs.at[j + 1]], buf1, sem1).wait()
                pltpu.sync_copy(buf1, acc.at[idxd.at[j + 1]], add=True)
                pltpu.async_copy(g_hbm.at[idxs.at[j + 3]], buf1, sem1)

            pltpu.make_async_copy(g_hbm.at[idxs.at[nb - 2]], buf0, sem0).wait()
            pltpu.sync_copy(buf0, acc.at[idxd.at[nb - 2]], add=True)
            pltpu.make_async_copy(g_hbm.at[idxs.at[nb - 1]], buf1, sem1).wait()
            pltpu.sync_copy(buf1, acc.at[idxd.at[nb - 1]], add=True)

        @pl.when(c == 0)
        def _():
            run(nb0, s * nb0)

        @pl.when(c == 1)
        def _():
            run(nb1, _NS * nb0 + s * nb1)

        plsc.subcore_barrier()
        for k in range(nz):
            off = s * rpt + k * _B
            pltpu.sync_copy(acc.at[pl.ds(off, _B)], out.at[c, pl.ds(off, _B)])

    return scat_kernel


def _dinv_block(dp_ref):
    deg = dp_ref[0, :, 0:1] + dp_ref[1, :, 0:1] + 1.0
    return lax.rsqrt(deg)


def _in_body(dp_ref, x_ref, w_ref, o_ref):
    dinv = _dinv_block(dp_ref)
    o_ref[...] = dinv * jnp.dot(x_ref[...], w_ref[...],
                                preferred_element_type=jnp.float32)


def _mid_body(dp_ref, sp_ref, g_ref, b_ref, w_ref, o_ref):
    dinv = _dinv_block(dp_ref)
    sfull = sp_ref[0] + sp_ref[1] + g_ref[...]
    h = jnp.maximum(dinv * sfull + b_ref[...], 0.0)
    o_ref[...] = dinv * jnp.dot(h, w_ref[...],
                                preferred_element_type=jnp.float32)


def _mid2_body(dp_ref, spa_ref, spb_ref, g_ref, b_ref, w_ref, o_ref):
    dinv = _dinv_block(dp_ref)
    sfull = jnp.concatenate(
        [spa_ref[0] + spa_ref[1], spb_ref[0] + spb_ref[1]], axis=1)
    sfull = sfull + g_ref[...]
    h = jnp.maximum(dinv * sfull + b_ref[...], 0.0)
    o_ref[...] = dinv * jnp.dot(h, w_ref[...],
                                preferred_element_type=jnp.float32)


def _out_body(dp_ref, sp_ref, g_ref, b_ref, o_ref):
    dinv = _dinv_block(dp_ref)
    sfull = sp_ref[0] + sp_ref[1] + g_ref[...]
    o_ref[...] = dinv * sfull + b_ref[...]


def _dp_spec():
    return pl.BlockSpec((2, _RB, 8), lambda i: (0, i, 0))


def _rows(d):
    return pl.BlockSpec((_RB, d), lambda i: (i, 0))


def _full(shape):
    nd = len(shape)
    return pl.BlockSpec(shape, lambda i: (0,) * nd)


def _tc_in(n, deg_parts, x, w):
    d_in, d_out = w.shape
    return pl.pallas_call(
        _in_body,
        grid=(n // _RB,),
        in_specs=[_dp_spec(), _rows(d_in), _full((d_in, d_out))],
        out_specs=_rows(d_out),
        out_shape=jax.ShapeDtypeStruct((n, d_out), jnp.float32),
    )(deg_parts, x, w)


def _tc_mid(n, deg_parts, s_parts, g, b, w):
    d_in, d_out = w.shape
    return pl.pallas_call(
        _mid_body,
        grid=(n // _RB,),
        in_specs=[_dp_spec(),
                  pl.BlockSpec((2, _RB, d_in), lambda i: (0, i, 0)),
                  _rows(d_in), _full((1, d_in)), _full((d_in, d_out))],
        out_specs=_rows(d_out),
        out_shape=jax.ShapeDtypeStruct((n, d_out), jnp.float32),
    )(deg_parts, s_parts, g, b.reshape(1, d_in), w)


def _tc_mid2(n, deg_parts, s_a, s_b, g, b, w):
    d_in, d_out = w.shape
    dh = d_in // 2
    half = pl.BlockSpec((2, _RB, dh), lambda i: (0, i, 0))
    return pl.pallas_call(
        _mid2_body,
        grid=(n // _RB,),
        in_specs=[_dp_spec(), half, half,
                  _rows(d_in), _full((1, d_in)), _full((d_in, d_out))],
        out_specs=_rows(d_out),
        out_shape=jax.ShapeDtypeStruct((n, d_out), jnp.float32),
    )(deg_parts, s_a, s_b, g, b.reshape(1, d_in), w)


def _tc_out(n, deg_parts, s_parts, g, b):
    d = g.shape[1]
    return pl.pallas_call(
        _out_body,
        grid=(n // _RB,),
        in_specs=[_dp_spec(),
                  pl.BlockSpec((2, _RB, d), lambda i: (0, i, 0)),
                  _rows(d), _full((1, d))],
        out_specs=_rows(d),
        out_shape=jax.ShapeDtypeStruct((n, d), jnp.float32),
    )(deg_parts, s_parts, g, b.reshape(1, d))


def kernel(x, edge_index, W1, b1, W2, b2, W3, b3):
    n, d_in = x.shape
    e = edge_index.shape[1]

    n_pad = _cdiv(n, _NS * _B) * _NS * _B
    tpp = _cdiv(e, _NS * _B)
    tpp += tpp % 2
    nb0 = max(4, 2 * int(tpp * 0.775 / 2))
    nb1 = max(4, tpp - nb0)
    nb0 = tpp - nb1
    tb = _NS * (nb0 + nb1)
    e_pad = tb * _B

    src = edge_index[0]
    dst = edge_index[1]
    pad = e_pad - e
    srcp = jnp.concatenate(
        [src, jnp.zeros((pad,), jnp.int32)]).reshape(tb, _B)
    dstp = jnp.concatenate(
        [dst, jnp.full((pad,), n, jnp.int32)]).reshape(tb, _B)

    ones8 = jnp.ones((_B, 8), jnp.float32)
    zeros8 = jnp.zeros((_B, 8), jnp.float32)
    dh = W1.shape[1] // 2
    z64 = jnp.zeros((_B, dh), jnp.float32)

    deg_parts = _make_deg_kernel(n_pad, tb)(dstp, ones8, zeros8)

    scat64 = _make_scatter_kernel(n, n_pad, nb0, nb1, dh)
    g1 = _tc_in(n, deg_parts, x, W1)
    s1a = scat64(g1[:, :dh], srcp, dstp, z64)
    s1b = scat64(g1[:, dh:], srcp, dstp, z64)
    g2 = _tc_mid2(n, deg_parts, s1a, s1b, g1, b1, W2)
    s2 = scat64(g2, srcp, dstp, z64)
    g3 = _tc_mid(n, deg_parts, s2, g2, b2, W3)
    s3 = scat64(g3, srcp, dstp, z64)
    return _tc_out(n, deg_parts, s3, g3, b3)
